# baseline (speedup 1.0000x reference)
import functools

import jax
import jax.numpy as jnp
from jax import lax
from jax.experimental import pallas as pl
from jax.experimental.pallas import tpu as pltpu
from jax.experimental.pallas import tpu_sc as plsc

B = 16384
D = 64
WIDE = 2 * D
NUM_ROWS_WIDE = 500000
NC = 2
NS = 16
NW = NC * NS
BPW = B // NW
WPW = BPW // 2
IDXW = 128
NCHUNK = BPW // IDXW
LANES = 16

_mesh = plsc.VectorSubcoreMesh(core_axis_name="c", subcore_axis_name="s")


@functools.partial(
    pl.kernel,
    mesh=_mesh,
    out_type=jax.ShapeDtypeStruct((B // 2, WIDE), jnp.float32),
    scratch_types=[
        pltpu.VMEM((NCHUNK, IDXW), jnp.int32),
        pltpu.VMEM((NCHUNK, IDXW), jnp.int32),
        pltpu.VMEM((BPW, WIDE), jnp.float32),
        pltpu.VMEM((WPW, WIDE), jnp.float32),
        pltpu.SemaphoreType.DMA,
    ],
)
def _embed_add(ctx_hbm, labels_hbm, table_hbm, out_hbm, lab_v, hi_v, rows_v,
               ctx_v, sem):
    wid = lax.axis_index("s") * NC + lax.axis_index("c")
    base = wid * WPW

    pltpu.sync_copy(labels_hbm.at[wid], lab_v)
    for j in range(NCHUNK):
        for c in range(IDXW // LANES):
            sl = pl.ds(c * LANES, LANES)
            hi_v[j, sl] = lab_v[j, sl] >> 1

    copies = [
        pltpu.async_copy(
            table_hbm.at[hi_v.at[j]], rows_v.at[pl.ds(j * IDXW, IDXW)], sem
        )
        for j in range(NCHUNK)
    ]
    pltpu.sync_copy(ctx_hbm.at[pl.ds(base, WPW)], ctx_v)
    for c in copies:
        c.wait()

    def body(g, carry):
        lab16 = lab_v[g // 8, pl.ds((g % 8) * LANES, LANES)]
        for i in range(LANES):
            off = (lab16[i] & 1) * D
            r = g * LANES + i
            rw = g * 8 + i // 2
            h = i % 2
            for c in range(D // LANES):
                dst = pl.ds(h * D + c * LANES, LANES)
                src = pl.ds(off + c * LANES, LANES)
                ctx_v[rw, dst] = ctx_v[rw, dst] + rows_v[r, src]
        return carry

    lax.fori_loop(0, BPW // LANES, body, 0)

    pltpu.sync_copy(ctx_v, out_hbm.at[pl.ds(base, WPW)])


def kernel(ctx_vec, labels, emb_weight):
    labels_r = labels.astype(jnp.int32).reshape(NW, NCHUNK, IDXW)
    ctx_wide = ctx_vec.reshape(B // 2, WIDE)
    table_wide = emb_weight.reshape(NUM_ROWS_WIDE, WIDE)
    out = _embed_add(ctx_wide, labels_r, table_wide)
    return out.reshape(B, D)


# speedup vs baseline: 1.5147x; 1.5147x over previous
import functools

import jax
import jax.numpy as jnp
from jax import lax
from jax.experimental import pallas as pl
from jax.experimental.pallas import tpu as pltpu
from jax.experimental.pallas import tpu_sc as plsc

B = 16384
D = 64
WIDE = 2 * D
NC = 2
NS = 16
NW = NC * NS
BPW = B // NW
WPW = BPW // 2
IDXW = 128
PASSW = 64
NPASS = BPW // PASSW
PROWS = PASSW * 8
LANES = 16

_mesh = plsc.VectorSubcoreMesh(core_axis_name="c", subcore_axis_name="s")


@functools.partial(
    pl.kernel,
    mesh=_mesh,
    out_type=jax.ShapeDtypeStruct((B // 2, WIDE), jnp.float32),
    scratch_types=[
        pltpu.VMEM((BPW // IDXW, IDXW), jnp.int32),
        pltpu.VMEM((PROWS, D), jnp.float32),
        pltpu.VMEM((WPW, WIDE), jnp.float32),
        pltpu.SemaphoreType.DMA,
        pltpu.SemaphoreType.DMA,
    ],
)
def _embed_add(ctx_hbm, labels_hbm, table_hbm, out_hbm, lab_v, gat_v, ctx_v,
               gsem, csem):
    wid = lax.axis_index("s") * NC + lax.axis_index("c")
    base = wid * WPW

    pltpu.sync_copy(labels_hbm.at[wid], lab_v)
    ctx_cp = pltpu.async_copy(ctx_hbm.at[pl.ds(base, WPW)], ctx_v, csem)
    ctx_cp.wait()

    for p in range(NPASS):
        jrow = p // 2
        jcol = (p % 2) * PASSW

        def fire(g, carry, jrow=jrow, jcol=jcol):
            lab16 = lab_v[jrow, pl.ds(jcol + g * LANES, LANES)]
            for i in range(LANES):
                grp = (lab16[i] >> 3) * 8
                pltpu.make_async_copy(
                    table_hbm.at[pl.ds(grp, 8)],
                    gat_v.at[pl.ds((g * LANES + i) * 8, 8)],
                    gsem,
                ).start()
            return carry

        lax.fori_loop(0, PASSW // LANES, fire, 0)
        pltpu.make_async_copy(table_hbm.at[pl.ds(0, PROWS)], gat_v, gsem).wait()

        def extract(g, carry, p=p, jrow=jrow, jcol=jcol):
            lab16 = lab_v[jrow, pl.ds(jcol + g * LANES, LANES)]
            for i in range(LANES):
                row = (g * LANES + i) * 8 + (lab16[i] & 7)
                rw = p * (PASSW // 2) + g * 8 + i // 2
                h = i % 2
                for c in range(D // LANES):
                    dst = pl.ds(h * D + c * LANES, LANES)
                    src = pl.ds(c * LANES, LANES)
                    ctx_v[rw, dst] = ctx_v[rw, dst] + gat_v[row, src]
            return carry

        lax.fori_loop(0, PASSW // LANES, extract, 0)

    pltpu.sync_copy(ctx_v, out_hbm.at[pl.ds(base, WPW)])


def kernel(ctx_vec, labels, emb_weight):
    labels_r = labels.astype(jnp.int32).reshape(NW, BPW // IDXW, IDXW)
    ctx_wide = ctx_vec.reshape(B // 2, WIDE)
    out = _embed_add(ctx_wide, labels_r, emb_weight)
    return out.reshape(B, D)


# speedup vs baseline: 2.1827x; 1.4410x over previous
import functools

import jax
import jax.numpy as jnp
from jax import lax
from jax.experimental import pallas as pl
from jax.experimental.pallas import tpu as pltpu
from jax.experimental.pallas import tpu_sc as plsc

B = 16384
D = 64
WIDE = 2 * D
NC = 2
NS = 16
NW = NC * NS
BPW = B // NW
WPW = BPW // 2
IDXW = 128
PASSW = 64
NPASS = BPW // PASSW
PROWS = PASSW * 8
LANES = 16

_mesh = plsc.VectorSubcoreMesh(core_axis_name="c", subcore_axis_name="s")


@functools.partial(
    pl.kernel,
    mesh=_mesh,
    out_type=jax.ShapeDtypeStruct((B // 2, WIDE), jnp.float32),
    scratch_types=[
        pltpu.VMEM((BPW // IDXW, IDXW), jnp.int32),
        pltpu.VMEM((PASSW, 8, D), jnp.float32),
        pltpu.VMEM((WPW, WIDE), jnp.float32),
        pltpu.SemaphoreType.DMA,
        pltpu.SemaphoreType.DMA,
    ],
)
def _embed_add(ctx_hbm, labels_hbm, table_hbm, out_hbm, lab_v, gat_v, ctx_v,
               gsem, csem):
    wid = lax.axis_index("s") * NC + lax.axis_index("c")
    base = wid * WPW

    pltpu.sync_copy(labels_hbm.at[wid], lab_v)
    ctx_cp = pltpu.async_copy(ctx_hbm.at[pl.ds(base, WPW)], ctx_v, csem)
    ctx_cp.wait()

    for p in range(NPASS):
        jrow = p // 2
        jcol = (p % 2) * PASSW

        def fire(g, carry, jrow=jrow, jcol=jcol):
            lab16 = lab_v[jrow, pl.ds(jcol + g * LANES, LANES)]
            for i in range(LANES):
                pltpu.make_async_copy(
                    table_hbm.at[lab16[i] >> 3],
                    gat_v.at[g * LANES + i],
                    gsem,
                ).start()
            return carry

        lax.fori_loop(0, PASSW // LANES, fire, 0)
        pltpu.make_async_copy(table_hbm.at[pl.ds(0, PASSW)], gat_v, gsem).wait()

        def extract(g, carry, p=p, jrow=jrow, jcol=jcol):
            lab16 = lab_v[jrow, pl.ds(jcol + g * LANES, LANES)]
            for i in range(LANES):
                s = lab16[i] & 7
                ti = g * LANES + i
                rw = p * (PASSW // 2) + g * 8 + i // 2
                h = i % 2
                for c in range(D // LANES):
                    dst = pl.ds(h * D + c * LANES, LANES)
                    src = pl.ds(c * LANES, LANES)
                    ctx_v[rw, dst] = ctx_v[rw, dst] + gat_v[ti, s, src]
            return carry

        lax.fori_loop(0, PASSW // LANES, extract, 0)

    pltpu.sync_copy(ctx_v, out_hbm.at[pl.ds(base, WPW)])


def kernel(ctx_vec, labels, emb_weight):
    labels_r = labels.astype(jnp.int32).reshape(NW, BPW // IDXW, IDXW)
    ctx_wide = ctx_vec.reshape(B // 2, WIDE)
    table_t = emb_weight.reshape(1000000 // 8, 8, D)
    out = _embed_add(ctx_wide, labels_r, table_t)
    return out.reshape(B, D)


# speedup vs baseline: 2.2105x; 1.0127x over previous
import functools

import jax
import jax.numpy as jnp
from jax import lax
from jax.experimental import pallas as pl
from jax.experimental.pallas import tpu as pltpu
from jax.experimental.pallas import tpu_sc as plsc

B = 16384
D = 64
WIDE = 2 * D
NC = 2
NS = 16
NW = NC * NS
BPW = B // NW
WPW = BPW // 2
IDXW = 128
PASSW = 32
NPASS = BPW // PASSW
NTILE = 1000000 // 8
LANES = 16

_mesh = plsc.VectorSubcoreMesh(core_axis_name="c", subcore_axis_name="s")


@functools.partial(
    pl.kernel,
    mesh=_mesh,
    out_type=jax.ShapeDtypeStruct((B // 2, WIDE), jnp.float32),
    scratch_types=[
        pltpu.VMEM((BPW // IDXW, IDXW), jnp.int32),
        pltpu.VMEM((PASSW, 8, D), jnp.float32),
        pltpu.VMEM((PASSW, 8, D), jnp.float32),
        pltpu.VMEM((WPW, WIDE), jnp.float32),
        pltpu.SemaphoreType.DMA,
        pltpu.SemaphoreType.DMA,
        pltpu.SemaphoreType.DMA,
    ],
)
def _embed_add(ctx_hbm, labels_hbm, table_hbm, out_hbm, lab_v, gat_a, gat_b,
               ctx_v, sem_a, sem_b, csem):
    wid = lax.axis_index("s") * NC + lax.axis_index("c")
    base = wid * WPW

    pltpu.sync_copy(labels_hbm.at[wid], lab_v)
    ctx_cp = pltpu.async_copy(ctx_hbm.at[pl.ds(base, WPW)], ctx_v, csem)

    bufs = (gat_a, gat_b)
    sems = (sem_a, sem_b)

    def fire(p):
        buf, sem = bufs[p % 2], sems[p % 2]
        jrow = p // 4
        jcol = (p % 4) * PASSW

        def body(g, carry):
            lab16 = lab_v[jrow, pl.ds(jcol + g * LANES, LANES)]
            for i in range(LANES):
                pltpu.make_async_copy(
                    table_hbm.at[lab16[i] >> 3],
                    buf.at[g * LANES + i],
                    sem,
                ).start()
            return carry

        lax.fori_loop(0, PASSW // LANES, body, 0)

    def extract(p):
        buf, sem = bufs[p % 2], sems[p % 2]
        pltpu.make_async_copy(table_hbm.at[pl.ds(0, PASSW)], buf, sem).wait()
        jrow = p // 4
        jcol = (p % 4) * PASSW

        def body(g, carry):
            lab16 = lab_v[jrow, pl.ds(jcol + g * LANES, LANES)]
            for i in range(LANES):
                s = lab16[i] & 7
                ti = g * LANES + i
                rw = p * (PASSW // 2) + g * 8 + i // 2
                h = i % 2
                for c in range(D // LANES):
                    dst = pl.ds(h * D + c * LANES, LANES)
                    src = pl.ds(c * LANES, LANES)
                    ctx_v[rw, dst] = ctx_v[rw, dst] + buf[ti, s, src]
            return carry

        lax.fori_loop(0, PASSW // LANES, body, 0)

    fire(0)
    ctx_cp.wait()
    for p in range(NPASS):
        if p + 1 < NPASS:
            fire(p + 1)
        extract(p)

    pltpu.sync_copy(ctx_v, out_hbm.at[pl.ds(base, WPW)])


def kernel(ctx_vec, labels, emb_weight):
    labels_r = labels.astype(jnp.int32).reshape(NW, BPW // IDXW, IDXW)
    ctx_wide = ctx_vec.reshape(B // 2, WIDE)
    table_t = emb_weight.reshape(NTILE, 8, D)
    out = _embed_add(ctx_wide, labels_r, table_t)
    return out.reshape(B, D)


# speedup vs baseline: 2.2198x; 1.0042x over previous
import functools

import jax
import jax.numpy as jnp
from jax import lax
from jax.experimental import pallas as pl
from jax.experimental.pallas import tpu as pltpu
from jax.experimental.pallas import tpu_sc as plsc

B = 16384
D = 64
WIDE = 2 * D
NC = 2
NS = 16
NW = NC * NS
BPW = B // NW
WPW = BPW // 2
IDXW = 128
PASSW = 32
NPASS = BPW // PASSW
NTILE = 1000000 // 8
LANES = 16

_mesh = plsc.VectorSubcoreMesh(core_axis_name="c", subcore_axis_name="s")


@functools.partial(
    pl.kernel,
    mesh=_mesh,
    out_type=jax.ShapeDtypeStruct((B // 2, WIDE), jnp.float32),
    scratch_types=[
        pltpu.VMEM((BPW // IDXW, IDXW), jnp.int32),
        pltpu.VMEM((PASSW, 8, D), jnp.float32),
        pltpu.VMEM((PASSW, 8, D), jnp.float32),
        pltpu.VMEM((WPW, WIDE), jnp.float32),
        pltpu.SemaphoreType.DMA,
        pltpu.SemaphoreType.DMA,
        pltpu.SemaphoreType.DMA,
    ],
)
def _embed_add(ctx_hbm, labels_hbm, table_hbm, out_hbm, lab_v, gat_a, gat_b,
               ctx_v, sem_a, sem_b, csem):
    wid = lax.axis_index("s") * NC + lax.axis_index("c")
    base = wid * WPW

    pltpu.sync_copy(labels_hbm.at[wid], lab_v)
    ctx_cp = pltpu.async_copy(ctx_hbm.at[pl.ds(base, WPW)], ctx_v, csem)

    bufs = (gat_a, gat_b)
    sems = (sem_a, sem_b)

    def fire(p):
        buf, sem = bufs[p % 2], sems[p % 2]
        jrow = p // 4
        jcol = (p % 4) * PASSW

        def body(g, carry):
            lab16 = lab_v[jrow, pl.ds(jcol + g * LANES, LANES)]
            for i in range(LANES):
                pltpu.make_async_copy(
                    table_hbm.at[lab16[i] >> 3],
                    buf.at[g * LANES + i],
                    sem,
                ).start()
            return carry

        lax.fori_loop(0, PASSW // LANES, body, 0)

    def extract(p):
        buf, sem = bufs[p % 2], sems[p % 2]
        pltpu.make_async_copy(table_hbm.at[pl.ds(0, PASSW)], buf, sem).wait()
        jrow = p // 4
        jcol = (p % 4) * PASSW

        def body(g, carry):
            lab16 = lab_v[jrow, pl.ds(jcol + g * LANES, LANES)]
            for i in range(LANES):
                s = lab16[i] & 7
                ti = g * LANES + i
                rw = p * (PASSW // 2) + g * 8 + i // 2
                h = i % 2
                for c in range(D // LANES):
                    dst = pl.ds(h * D + c * LANES, LANES)
                    src = pl.ds(c * LANES, LANES)
                    plsc.addupdate(ctx_v.at[rw, dst], buf[ti, s, src])
            return carry

        lax.fori_loop(0, PASSW // LANES, body, 0)

    fire(0)
    ctx_cp.wait()
    for p in range(NPASS):
        if p + 1 < NPASS:
            fire(p + 1)
        extract(p)

    pltpu.sync_copy(ctx_v, out_hbm.at[pl.ds(base, WPW)])


def kernel(ctx_vec, labels, emb_weight):
    labels_r = labels.astype(jnp.int32).reshape(NW, BPW // IDXW, IDXW)
    ctx_wide = ctx_vec.reshape(B // 2, WIDE)
    table_t = emb_weight.reshape(NTILE, 8, D)
    out = _embed_add(ctx_wide, labels_r, table_t)
    return out.reshape(B, D)


# speedup vs baseline: 2.3157x; 1.0432x over previous
import functools

import jax
import jax.numpy as jnp
from jax import lax
from jax.experimental import pallas as pl
from jax.experimental.pallas import tpu as pltpu
from jax.experimental.pallas import tpu_sc as plsc

B = 16384
D = 64
NC = 2
NS = 16
NW = NC * NS
BPW = B // NW
IDXW = 128
PASSW = 16
NPASS = BPW // PASSW
NTILE = 1000000 // 8
LANES = 16

_mesh = plsc.VectorSubcoreMesh(core_axis_name="c", subcore_axis_name="s")


@functools.partial(
    pl.kernel,
    mesh=_mesh,
    out_type=jax.ShapeDtypeStruct((B, D), jnp.float32),
    scratch_types=[
        pltpu.VMEM((BPW // IDXW, IDXW), jnp.int32),
        pltpu.VMEM((PASSW, 8, D), jnp.float32),
        pltpu.VMEM((PASSW, 8, D), jnp.float32),
        pltpu.VMEM((BPW, D), jnp.float32),
        pltpu.SemaphoreType.DMA,
        pltpu.SemaphoreType.DMA,
        pltpu.SemaphoreType.DMA,
    ],
)
def _embed_add(ctx_hbm, labels_hbm, table_hbm, out_hbm, lab_v, gat_a, gat_b,
               ctx_v, sem_a, sem_b, csem):
    wid = lax.axis_index("s") * NC + lax.axis_index("c")
    base = wid * BPW

    pltpu.sync_copy(labels_hbm.at[wid], lab_v)
    ctx_cp = pltpu.async_copy(ctx_hbm.at[pl.ds(base, BPW)], ctx_v, csem)

    def lab16_of(p):
        return lab_v[p // 8, pl.ds((p % 8) * PASSW, LANES)]

    def fire(p, buf, sem):
        lab16 = lab16_of(p)
        for i in range(LANES):
            pltpu.make_async_copy(
                table_hbm.at[lab16[i] >> 3], buf.at[i], sem
            ).start()

    def extract(p, buf, sem):
        pltpu.make_async_copy(table_hbm.at[pl.ds(0, PASSW)], buf, sem).wait()
        lab16 = lab16_of(p)
        for i in range(LANES):
            s = lab16[i] & 7
            r = p * PASSW + i
            for c in range(D // LANES):
                sl = pl.ds(c * LANES, LANES)
                plsc.addupdate(ctx_v.at[r, sl], buf[i, s, sl])

    fire(0, gat_a, sem_a)
    ctx_cp.wait()

    def body(k, carry):
        p0 = 2 * k
        fire(p0 + 1, gat_b, sem_b)
        extract(p0, gat_a, sem_a)

        @pl.when(p0 + 2 < NPASS)
        def _():
            fire(p0 + 2, gat_a, sem_a)

        extract(p0 + 1, gat_b, sem_b)
        return carry

    lax.fori_loop(0, NPASS // 2, body, 0)

    pltpu.sync_copy(ctx_v, out_hbm.at[pl.ds(base, BPW)])


def kernel(ctx_vec, labels, emb_weight):
    labels_r = labels.astype(jnp.int32).reshape(NW, BPW // IDXW, IDXW)
    table_t = emb_weight.reshape(NTILE, 8, D)
    return _embed_add(ctx_vec, labels_r, table_t)
